# baseline (speedup 1.0000x reference)
import functools

import jax
import jax.numpy as jnp
from jax import lax
from jax.experimental import pallas as pl
from jax.experimental.pallas import tpu as pltpu
from jax.experimental.pallas import tpu_sc as plsc

B, T, D, F, E, K = 2, 2048, 1024, 4096, 8, 2
N = B * T
NK = N * K
BM = 256
P = NK + E * BM
NBLK = P // BM
BF = 512

NC, NS, L = 2, 16, 16
NW = NC * NS
RPW = P // NW
RCH = RPW // 4
assert RPW % (4 * 8) == 0 and RCH <= 128
TPW = N // NW
TCH = 32

GR = 128
NG = N // GR


def _router_kernel(x_ref, wr_ref, pos_ref, wts_ref, be_ref):
    x = x_ref[...]
    logits = lax.dot_general(
        x, wr_ref[...], (((1,), (1,)), ((), ())),
        preferred_element_type=jnp.float32)
    i0 = jnp.argmax(logits, axis=-1)
    m0 = jnp.max(logits, axis=-1)
    eids = lax.broadcasted_iota(jnp.int32, (N, E), 1)
    masked = jnp.where(eids == i0[:, None], -jnp.inf, logits)
    i1 = jnp.argmax(masked, axis=-1)
    m1 = jnp.max(masked, axis=-1)
    w0 = 1.0 / (1.0 + jnp.exp(m1 - m0))
    M0 = (eids == i0[:, None]).astype(jnp.float32)
    M1 = (eids == i1[:, None]).astype(jnp.float32)
    Msum = M0 + M1

    a_ids = lax.broadcasted_iota(jnp.int32, (GR, GR), 1)
    b_ids = lax.broadcasted_iota(jnp.int32, (GR, GR), 0)
    tlow = (a_ids < b_ids).astype(jnp.float32)
    run = jnp.zeros((1, E), jnp.float32)
    c_parts = []
    for g in range(NG):
        chunk = Msum[g * GR:(g + 1) * GR, :]
        pin = lax.dot_general(
            tlow, chunk, (((1,), (0,)), ((), ())),
            preferred_element_type=jnp.float32)
        c_parts.append(pin + run)
        run = run + jnp.sum(chunk, axis=0, keepdims=True)
    C = jnp.concatenate(c_parts, axis=0)
    counts = run

    padded = jnp.ceil(counts * (1.0 / BM)) * BM
    e_a = lax.broadcasted_iota(jnp.int32, (E, E), 0)
    e_b = lax.broadcasted_iota(jnp.int32, (E, E), 1)
    tlow8 = (e_a < e_b).astype(jnp.float32)
    off = lax.dot_general(
        padded, tlow8, (((1,), (0,)), ((), ())),
        preferred_element_type=jnp.float32)
    dstf = off + C
    dst0 = jnp.sum(M0 * dstf, axis=1).astype(jnp.int32)
    dst1 = jnp.sum(M1 * dstf, axis=1).astype(jnp.int32)
    pos_ref[...] = jnp.concatenate(
        [dst0[:, None], dst1[:, None]], axis=1)
    wts_ref[...] = jnp.concatenate(
        [w0[:, None], (1.0 - w0)[:, None]], axis=1)

    off_end = off + padded
    blk = (lax.broadcasted_iota(jnp.int32, (NBLK, E), 0) * BM
           ).astype(jnp.float32)
    be = jnp.sum((blk >= off_end).astype(jnp.int32), axis=1, keepdims=True)
    be_ref[...] = jnp.minimum(be, E - 1)


def _gmm_kernel(be_ref, xs_ref, w1_ref, w2_ref, ys_ref):
    f = pl.program_id(0)
    b = pl.program_id(1)
    xb = xs_ref[...].astype(jnp.bfloat16)
    h = lax.dot_general(
        xb, w1_ref[0], (((1,), (1,)), ((), ())),
        preferred_element_type=jnp.float32)
    h = 0.5 * h * (1.0 + lax.erf(h * 0.7071067811865476))
    contrib = lax.dot_general(
        h.astype(jnp.bfloat16), w2_ref[0], (((1,), (1,)), ((), ())),
        preferred_element_type=jnp.float32)
    row0 = b * BM

    @pl.when(f == 0)
    def _init():
        ys_ref[pl.ds(row0, BM), :] = contrib

    @pl.when(f != 0)
    def _acc():
        ys_ref[pl.ds(row0, BM), :] += contrib


def _sc_dispatch_body(xf_hbm, dst_hbm, xs_hbm, dst_v, ptok_v, rows_v, sem):
    wid = lax.axis_index("s") * NC + lax.axis_index("c")
    base = wid * RPW
    pltpu.sync_copy(dst_hbm, dst_v)

    zero16 = jnp.zeros((L,), jnp.int32)

    def _init(i, _):
        ptok_v[pl.ds(i * L, L)] = zero16
        return 0
    lax.fori_loop(0, RPW // L, _init, 0)

    lane = lax.broadcasted_iota(jnp.int32, (L,), 0)

    def _scan(i, _):
        d = dst_v[pl.ds(i * L, L)]
        tok = lax.shift_right_logical(lane + i * L, 1)
        loc = d - base
        m = jnp.logical_and(loc >= 0, loc < RPW)
        locc = jnp.clip(loc, 0, RPW - 1)
        plsc.store_scatter(ptok_v, [locc], tok, mask=m)
        return 0
    lax.fori_loop(0, NK // L, _scan, 0)

    for ci in range(RPW // RCH):
        idx = ptok_v.at[pl.ds(ci * RCH, RCH)]
        pltpu.async_copy(xf_hbm.at[idx], rows_v, sem).wait()
        pltpu.sync_copy(rows_v, xs_hbm.at[pl.ds(base + ci * RCH, RCH)])


def _sc_combine_body(ys_hbm, pos_hbm, wts_hbm, out_hbm,
                     pos_v, w_v, rows_v, out_v, sem):
    wid = lax.axis_index("s") * NC + lax.axis_index("c")
    lane = lax.broadcasted_iota(jnp.int32, (L,), 0)

    for ci in range(TPW // TCH):
        tb = wid * TPW + ci * TCH
        pltpu.sync_copy(pos_hbm.at[pl.ds(2 * tb, 2 * TCH)], pos_v)
        pltpu.sync_copy(wts_hbm.at[pl.ds(2 * tb, 2 * TCH)], w_v)
        pltpu.async_copy(ys_hbm.at[pos_v], rows_v, sem).wait()

        def _token(j, _):
            f0 = jnp.full((L,), 2 * j, jnp.int32)
            f1 = jnp.full((L,), 2 * j + 1, jnp.int32)
            w0 = plsc.load_gather(w_v, [f0])
            w1 = plsc.load_gather(w_v, [f1])
            fj = jnp.full((L,), j, jnp.int32)

            def _vec(v, _):
                cols = lane + v * L
                r0 = plsc.load_gather(rows_v, [f0, cols])
                r1 = plsc.load_gather(rows_v, [f1, cols])
                plsc.store_scatter(out_v, [fj, cols], w0 * r0 + w1 * r1)
                return 0
            lax.fori_loop(0, D // L, _vec, 0, unroll=8)
            return 0
        lax.fori_loop(0, TCH, _token, 0)
        pltpu.sync_copy(out_v, out_hbm.at[pl.ds(tb, TCH)])


@functools.lru_cache(maxsize=1)
def _make_sc_kernels():
    mesh = plsc.VectorSubcoreMesh(
        core_axis_name="c", subcore_axis_name="s",
        num_cores=NC, num_subcores=NS)
    sc_params = pltpu.CompilerParams(needs_layout_passes=False)
    dispatch = pl.kernel(
        _sc_dispatch_body,
        out_type=jax.ShapeDtypeStruct((P, D), jnp.float32),
        mesh=mesh,
        compiler_params=sc_params,
        scratch_types=[
            pltpu.VMEM((NK,), jnp.int32),
            pltpu.VMEM((RPW,), jnp.int32),
            pltpu.VMEM((RCH, D), jnp.float32),
            pltpu.SemaphoreType.DMA,
        ],
    )
    combine = pl.kernel(
        _sc_combine_body,
        out_type=jax.ShapeDtypeStruct((N, D), jnp.float32),
        mesh=mesh,
        compiler_params=sc_params,
        scratch_types=[
            pltpu.VMEM((2 * TCH,), jnp.int32),
            pltpu.VMEM((2 * TCH,), jnp.float32),
            pltpu.VMEM((2 * TCH, D), jnp.float32),
            pltpu.VMEM((TCH, D), jnp.float32),
            pltpu.SemaphoreType.DMA,
        ],
    )
    return dispatch, combine


@jax.jit
def kernel(x, Wr, W1, W2):
    xf = x.reshape(N, D)
    _sc_dispatch, _sc_combine = _make_sc_kernels()

    pos, wts, be = pl.pallas_call(
        _router_kernel,
        in_specs=[
            pl.BlockSpec((N, D), lambda: (0, 0)),
            pl.BlockSpec((E, D), lambda: (0, 0)),
        ],
        out_specs=[
            pl.BlockSpec((N, K), lambda: (0, 0)),
            pl.BlockSpec((N, K), lambda: (0, 0)),
            pl.BlockSpec((NBLK, 1), lambda: (0, 0)),
        ],
        out_shape=[
            jax.ShapeDtypeStruct((N, K), jnp.int32),
            jax.ShapeDtypeStruct((N, K), jnp.float32),
            jax.ShapeDtypeStruct((NBLK, 1), jnp.int32),
        ],
    )(xf, Wr)

    pos_flat = pos.reshape(NK)
    xs = _sc_dispatch(xf, pos_flat)

    ys = pl.pallas_call(
        _gmm_kernel,
        grid_spec=pltpu.PrefetchScalarGridSpec(
            num_scalar_prefetch=1,
            grid=(F // BF, NBLK),
            in_specs=[
                pl.BlockSpec((BM, D), lambda f, b, be: (b, 0)),
                pl.BlockSpec((1, BF, D), lambda f, b, be: (be[b, 0], f, 0)),
                pl.BlockSpec((1, D, BF), lambda f, b, be: (be[b, 0], 0, f)),
            ],
            out_specs=pl.BlockSpec((P, D), lambda f, b, be: (0, 0)),
        ),
        out_shape=jax.ShapeDtypeStruct((P, D), jnp.float32),
    )(be, xs, W1.astype(jnp.bfloat16), W2.astype(jnp.bfloat16))

    out = _sc_combine(ys, pos_flat, wts.reshape(NK))
    return out.reshape(B, T, D)


# speedup vs baseline: 1.1326x; 1.1326x over previous
import functools

import jax
import jax.numpy as jnp
from jax import lax
from jax.experimental import pallas as pl
from jax.experimental.pallas import tpu as pltpu
from jax.experimental.pallas import tpu_sc as plsc

B, T, D, F, E, K = 2, 2048, 1024, 4096, 8, 2
N = B * T
NK = N * K
BM = 256
P = NK + E * BM
NBLK = P // BM
BF = 512

NC, NS, L = 2, 16, 16
NW = NC * NS
RPW = P // NW
RCH = RPW // 8
assert RPW % (8 * 8) == 0 and RCH <= 128
TPW = N // NW
TCH = 16

GR = 128
NG = N // GR


def _router_kernel(x_ref, wr_ref, pos_ref, wts_ref, be_ref):
    x = x_ref[...]
    logits = lax.dot_general(
        x, wr_ref[...], (((1,), (1,)), ((), ())),
        preferred_element_type=jnp.float32)
    i0 = jnp.argmax(logits, axis=-1)
    m0 = jnp.max(logits, axis=-1)
    eids = lax.broadcasted_iota(jnp.int32, (N, E), 1)
    masked = jnp.where(eids == i0[:, None], -jnp.inf, logits)
    i1 = jnp.argmax(masked, axis=-1)
    m1 = jnp.max(masked, axis=-1)
    w0 = 1.0 / (1.0 + jnp.exp(m1 - m0))
    M0 = (eids == i0[:, None]).astype(jnp.float32)
    M1 = (eids == i1[:, None]).astype(jnp.float32)
    Msum = M0 + M1

    a_ids = lax.broadcasted_iota(jnp.int32, (GR, GR), 1)
    b_ids = lax.broadcasted_iota(jnp.int32, (GR, GR), 0)
    tlow = (a_ids < b_ids).astype(jnp.float32)
    run = jnp.zeros((1, E), jnp.float32)
    c_parts = []
    for g in range(NG):
        chunk = Msum[g * GR:(g + 1) * GR, :]
        pin = lax.dot_general(
            tlow, chunk, (((1,), (0,)), ((), ())),
            preferred_element_type=jnp.float32)
        c_parts.append(pin + run)
        run = run + jnp.sum(chunk, axis=0, keepdims=True)
    C = jnp.concatenate(c_parts, axis=0)
    counts = run

    padded = jnp.ceil(counts * (1.0 / BM)) * BM
    e_a = lax.broadcasted_iota(jnp.int32, (E, E), 0)
    e_b = lax.broadcasted_iota(jnp.int32, (E, E), 1)
    tlow8 = (e_a < e_b).astype(jnp.float32)
    off = lax.dot_general(
        padded, tlow8, (((1,), (0,)), ((), ())),
        preferred_element_type=jnp.float32)
    dstf = off + C
    dst0 = jnp.sum(M0 * dstf, axis=1).astype(jnp.int32)
    dst1 = jnp.sum(M1 * dstf, axis=1).astype(jnp.int32)
    pos_ref[...] = jnp.concatenate(
        [dst0[:, None], dst1[:, None]], axis=1)
    wts_ref[...] = jnp.concatenate(
        [w0[:, None], (1.0 - w0)[:, None]], axis=1)

    off_end = off + padded
    blk = (lax.broadcasted_iota(jnp.int32, (NBLK, E), 0) * BM
           ).astype(jnp.float32)
    be = jnp.sum((blk >= off_end).astype(jnp.int32), axis=1, keepdims=True)
    be_ref[...] = jnp.minimum(be, E - 1)


def _gmm_kernel(be_ref, xs_ref, w1_ref, w2_ref, ys_ref):
    f = pl.program_id(0)
    b = pl.program_id(1)
    xb = xs_ref[...]
    h = lax.dot_general(
        xb, w1_ref[0], (((1,), (1,)), ((), ())),
        preferred_element_type=jnp.float32)
    h = 0.5 * h * (1.0 + lax.erf(h * 0.7071067811865476))
    contrib = lax.dot_general(
        h, w2_ref[0], (((1,), (1,)), ((), ())),
        preferred_element_type=jnp.float32)
    row0 = b * BM

    @pl.when(f == 0)
    def _init():
        ys_ref[pl.ds(row0, BM), :] = contrib

    @pl.when(f != 0)
    def _acc():
        ys_ref[pl.ds(row0, BM), :] += contrib


def _sc_dispatch_body(xf_hbm, dst_hbm, xs_hbm, dst_v, ptok_v,
                      rows0_v, rows1_v, gsem0, gsem1, osem0, osem1):
    wid = lax.axis_index("s") * NC + lax.axis_index("c")
    base = wid * RPW
    pltpu.sync_copy(dst_hbm, dst_v)

    zero16 = jnp.zeros((L,), jnp.int32)

    def _init(i, _):
        ptok_v[pl.ds(i * L, L)] = zero16
        return 0
    lax.fori_loop(0, RPW // L, _init, 0, unroll=4)

    lane = lax.broadcasted_iota(jnp.int32, (L,), 0)

    def _scan(i, _):
        d = dst_v[pl.ds(i * L, L)]
        tok = lax.shift_right_logical(lane + i * L, 1)
        loc = d - base
        m = jnp.logical_and(loc >= 0, loc < RPW)
        locc = jnp.clip(loc, 0, RPW - 1)
        plsc.store_scatter(ptok_v, [locc], tok, mask=m)
        return 0
    lax.fori_loop(0, NK // L, _scan, 0, unroll=8)

    bufs = (rows0_v, rows1_v)
    gsems = (gsem0, gsem1)
    osems = (osem0, osem1)
    nch = RPW // RCH

    def _gather(ci, buf, sem):
        idx = ptok_v.at[pl.ds(ci * RCH, RCH)]
        return pltpu.async_copy(xf_hbm.at[idx], buf, sem)

    g_next = _gather(0, bufs[0], gsems[0])
    out_pending = [None, None]
    for ci in range(nch):
        b = ci % 2
        g_next.wait()
        if ci + 1 < nch:
            nb = 1 - b
            if out_pending[nb] is not None:
                out_pending[nb].wait()
                out_pending[nb] = None
            g_next = _gather(ci + 1, bufs[nb], gsems[nb])
        if out_pending[b] is not None:
            out_pending[b].wait()
            out_pending[b] = None
        out_pending[b] = pltpu.async_copy(
            bufs[b], xs_hbm.at[pl.ds(base + ci * RCH, RCH)], osems[b])
    for d in out_pending:
        if d is not None:
            d.wait()


def _sc_combine_body(ys_hbm, pos_hbm, wts_hbm, out_hbm,
                     pos0_v, pos1_v, w0_v, w1_v, rows0_v, rows1_v,
                     out0_v, out1_v, gsem0, gsem1, osem0, osem1):
    wid = lax.axis_index("s") * NC + lax.axis_index("c")
    lane = lax.broadcasted_iota(jnp.int32, (L,), 0)

    posb = (pos0_v, pos1_v)
    wb = (w0_v, w1_v)
    rows = (rows0_v, rows1_v)
    outs = (out0_v, out1_v)
    gsems = (gsem0, gsem1)
    osems = (osem0, osem1)
    nch = TPW // TCH

    def _gather(ci, b):
        tb = wid * TPW + ci * TCH
        pltpu.sync_copy(pos_hbm.at[pl.ds(2 * tb, 2 * TCH)], posb[b])
        pltpu.sync_copy(wts_hbm.at[pl.ds(2 * tb, 2 * TCH)], wb[b])
        return pltpu.async_copy(ys_hbm.at[posb[b]], rows[b], gsems[b])

    g_next = _gather(0, 0)
    out_pending = [None, None]
    for ci in range(nch):
        b = ci % 2
        g_next.wait()
        if ci + 1 < nch:
            g_next = _gather(ci + 1, 1 - b)
        if out_pending[b] is not None:
            out_pending[b].wait()
            out_pending[b] = None
        rows_v, w_v, out_v = rows[b], wb[b], outs[b]

        def _token(j, _):
            f0 = jnp.full((L,), 2 * j, jnp.int32)
            f1 = jnp.full((L,), 2 * j + 1, jnp.int32)
            w0 = plsc.load_gather(w_v, [f0])
            w1 = plsc.load_gather(w_v, [f1])
            fj = jnp.full((L,), j, jnp.int32)

            def _vec(v, _):
                cols = lane + v * L
                r0 = plsc.load_gather(rows_v, [f0, cols])
                r1 = plsc.load_gather(rows_v, [f1, cols])
                plsc.store_scatter(out_v, [fj, cols], w0 * r0 + w1 * r1)
                return 0
            lax.fori_loop(0, D // L, _vec, 0, unroll=8)
            return 0
        lax.fori_loop(0, TCH, _token, 0)
        tb = wid * TPW + ci * TCH
        out_pending[b] = pltpu.async_copy(
            out_v, out_hbm.at[pl.ds(tb, TCH)], osems[b])
    for d in out_pending:
        if d is not None:
            d.wait()


@functools.lru_cache(maxsize=1)
def _make_sc_kernels():
    mesh = plsc.VectorSubcoreMesh(
        core_axis_name="c", subcore_axis_name="s",
        num_cores=NC, num_subcores=NS)
    sc_params = pltpu.CompilerParams(needs_layout_passes=False)
    dispatch = pl.kernel(
        _sc_dispatch_body,
        out_type=jax.ShapeDtypeStruct((P, D), jnp.float32),
        mesh=mesh,
        compiler_params=sc_params,
        scratch_types=[
            pltpu.VMEM((NK,), jnp.int32),
            pltpu.VMEM((RPW,), jnp.int32),
            pltpu.VMEM((RCH, D), jnp.float32),
            pltpu.VMEM((RCH, D), jnp.float32),
            pltpu.SemaphoreType.DMA,
            pltpu.SemaphoreType.DMA,
            pltpu.SemaphoreType.DMA,
            pltpu.SemaphoreType.DMA,
        ],
    )
    combine = pl.kernel(
        _sc_combine_body,
        out_type=jax.ShapeDtypeStruct((N, D), jnp.float32),
        mesh=mesh,
        compiler_params=sc_params,
        scratch_types=[
            pltpu.VMEM((2 * TCH,), jnp.int32),
            pltpu.VMEM((2 * TCH,), jnp.int32),
            pltpu.VMEM((2 * TCH,), jnp.float32),
            pltpu.VMEM((2 * TCH,), jnp.float32),
            pltpu.VMEM((2 * TCH, D), jnp.float32),
            pltpu.VMEM((2 * TCH, D), jnp.float32),
            pltpu.VMEM((TCH, D), jnp.float32),
            pltpu.VMEM((TCH, D), jnp.float32),
            pltpu.SemaphoreType.DMA,
            pltpu.SemaphoreType.DMA,
            pltpu.SemaphoreType.DMA,
            pltpu.SemaphoreType.DMA,
        ],
    )
    return dispatch, combine


@jax.jit
def kernel(x, Wr, W1, W2):
    xf = x.reshape(N, D)
    _sc_dispatch, _sc_combine = _make_sc_kernels()

    pos, wts, be = pl.pallas_call(
        _router_kernel,
        in_specs=[
            pl.BlockSpec((N, D), lambda: (0, 0)),
            pl.BlockSpec((E, D), lambda: (0, 0)),
        ],
        out_specs=[
            pl.BlockSpec((N, K), lambda: (0, 0)),
            pl.BlockSpec((N, K), lambda: (0, 0)),
            pl.BlockSpec((NBLK, 1), lambda: (0, 0)),
        ],
        out_shape=[
            jax.ShapeDtypeStruct((N, K), jnp.int32),
            jax.ShapeDtypeStruct((N, K), jnp.float32),
            jax.ShapeDtypeStruct((NBLK, 1), jnp.int32),
        ],
    )(xf, Wr)

    pos_flat = pos.reshape(NK)
    xs = _sc_dispatch(xf, pos_flat)

    ys = pl.pallas_call(
        _gmm_kernel,
        grid_spec=pltpu.PrefetchScalarGridSpec(
            num_scalar_prefetch=1,
            grid=(F // BF, NBLK),
            in_specs=[
                pl.BlockSpec((BM, D), lambda f, b, be: (b, 0)),
                pl.BlockSpec((1, BF, D), lambda f, b, be: (be[b, 0], f, 0)),
                pl.BlockSpec((1, D, BF), lambda f, b, be: (be[b, 0], 0, f)),
            ],
            out_specs=pl.BlockSpec((P, D), lambda f, b, be: (0, 0)),
        ),
        out_shape=jax.ShapeDtypeStruct((P, D), jnp.float32),
    )(be, xs, W1, W2)

    out = _sc_combine(ys, pos_flat, wts.reshape(NK))
    return out.reshape(B, T, D)


# speedup vs baseline: 1.3747x; 1.2138x over previous
import functools

import jax
import jax.numpy as jnp
from jax import lax
from jax.experimental import pallas as pl
from jax.experimental.pallas import tpu as pltpu
from jax.experimental.pallas import tpu_sc as plsc

B, T, D, F, E, K = 2, 2048, 1024, 4096, 8, 2
N = B * T
NK = N * K
BM = 256
P = NK + E * BM
NBLK = P // BM
BF = 1024

NC, NS, L = 2, 16, 16
NW = NC * NS
RPW = P // NW
RCH = RPW // 8
assert RPW % (8 * 8) == 0 and RCH <= 128
TPW = N // NW
TCH = 16

GR = 128
NG = N // GR


def _router_kernel(x_ref, wr_ref, pos_ref, wts_ref, be_ref):
    x = x_ref[...]
    logits = lax.dot_general(
        x, wr_ref[...], (((1,), (1,)), ((), ())),
        preferred_element_type=jnp.float32)
    i0 = jnp.argmax(logits, axis=-1)
    m0 = jnp.max(logits, axis=-1)
    eids = lax.broadcasted_iota(jnp.int32, (N, E), 1)
    masked = jnp.where(eids == i0[:, None], -jnp.inf, logits)
    i1 = jnp.argmax(masked, axis=-1)
    m1 = jnp.max(masked, axis=-1)
    w0 = 1.0 / (1.0 + jnp.exp(m1 - m0))
    M0 = (eids == i0[:, None]).astype(jnp.float32)
    M1 = (eids == i1[:, None]).astype(jnp.float32)
    Msum = M0 + M1

    a_ids = lax.broadcasted_iota(jnp.int32, (GR, GR), 1)
    b_ids = lax.broadcasted_iota(jnp.int32, (GR, GR), 0)
    tlow = (a_ids < b_ids).astype(jnp.float32)
    run = jnp.zeros((1, E), jnp.float32)
    c_parts = []
    for g in range(NG):
        chunk = Msum[g * GR:(g + 1) * GR, :]
        pin = lax.dot_general(
            tlow, chunk, (((1,), (0,)), ((), ())),
            preferred_element_type=jnp.float32)
        c_parts.append(pin + run)
        run = run + jnp.sum(chunk, axis=0, keepdims=True)
    C = jnp.concatenate(c_parts, axis=0)
    counts = run

    padded = jnp.ceil(counts * (1.0 / BM)) * BM
    e_a = lax.broadcasted_iota(jnp.int32, (E, E), 0)
    e_b = lax.broadcasted_iota(jnp.int32, (E, E), 1)
    tlow8 = (e_a < e_b).astype(jnp.float32)
    off = lax.dot_general(
        padded, tlow8, (((1,), (0,)), ((), ())),
        preferred_element_type=jnp.float32)
    dstf = off + C
    dst0 = jnp.sum(M0 * dstf, axis=1).astype(jnp.int32)
    dst1 = jnp.sum(M1 * dstf, axis=1).astype(jnp.int32)
    pos_ref[...] = jnp.concatenate(
        [dst0[:, None], dst1[:, None]], axis=1)
    wts_ref[...] = jnp.concatenate(
        [w0[:, None], (1.0 - w0)[:, None]], axis=1)

    off_end = off + padded
    blk = (lax.broadcasted_iota(jnp.int32, (NBLK, E), 0) * BM
           ).astype(jnp.float32)
    be = jnp.sum((blk >= off_end).astype(jnp.int32), axis=1, keepdims=True)
    be_ref[...] = jnp.minimum(be, E - 1)


def _gmm_kernel(be_ref, xs_ref, w1_ref, w2_ref, ys_ref):
    f = pl.program_id(0)
    b = pl.program_id(1)
    xb = xs_ref[...]
    h = lax.dot_general(
        xb, w1_ref[0], (((1,), (1,)), ((), ())),
        preferred_element_type=jnp.float32)
    h = 0.5 * h * (1.0 + lax.erf(h * 0.7071067811865476))
    contrib = lax.dot_general(
        h, w2_ref[0], (((1,), (1,)), ((), ())),
        preferred_element_type=jnp.float32)
    row0 = b * BM

    @pl.when(f == 0)
    def _init():
        ys_ref[pl.ds(row0, BM), :] = contrib

    @pl.when(f != 0)
    def _acc():
        ys_ref[pl.ds(row0, BM), :] += contrib


def _sc_dispatch_body(xf_hbm, dst_hbm, xs_hbm, dst_v, ptok_v,
                      rows0_v, rows1_v, gsem0, gsem1, osem0, osem1):
    wid = lax.axis_index("s") * NC + lax.axis_index("c")
    base = wid * RPW
    pltpu.sync_copy(dst_hbm, dst_v)

    zero16 = jnp.zeros((L,), jnp.int32)

    def _init(i, _):
        ptok_v[pl.ds(i * L, L)] = zero16
        return 0
    lax.fori_loop(0, RPW // L, _init, 0, unroll=4)

    lane = lax.broadcasted_iota(jnp.int32, (L,), 0)

    def _scan(i, _):
        d = dst_v[pl.ds(i * L, L)]
        tok = lax.shift_right_logical(lane + i * L, 1)
        loc = d - base
        m = jnp.logical_and(loc >= 0, loc < RPW)
        locc = jnp.clip(loc, 0, RPW - 1)
        plsc.store_scatter(ptok_v, [locc], tok, mask=m)
        return 0
    lax.fori_loop(0, NK // L, _scan, 0, unroll=8)

    bufs = (rows0_v, rows1_v)
    gsems = (gsem0, gsem1)
    osems = (osem0, osem1)
    nch = RPW // RCH

    def _gather(ci, buf, sem):
        idx = ptok_v.at[pl.ds(ci * RCH, RCH)]
        return pltpu.async_copy(xf_hbm.at[idx], buf, sem)

    g_next = _gather(0, bufs[0], gsems[0])
    out_pending = [None, None]
    for ci in range(nch):
        b = ci % 2
        g_next.wait()
        if ci + 1 < nch:
            nb = 1 - b
            if out_pending[nb] is not None:
                out_pending[nb].wait()
                out_pending[nb] = None
            g_next = _gather(ci + 1, bufs[nb], gsems[nb])
        if out_pending[b] is not None:
            out_pending[b].wait()
            out_pending[b] = None
        out_pending[b] = pltpu.async_copy(
            bufs[b], xs_hbm.at[pl.ds(base + ci * RCH, RCH)], osems[b])
    for d in out_pending:
        if d is not None:
            d.wait()


def _sc_combine_body(ys_hbm, pos_hbm, wts_hbm, out_hbm,
                     pos0_v, pos1_v, w0_v, w1_v, rows0_v, rows1_v,
                     out0_v, out1_v, gsem0, gsem1, osem0, osem1):
    wid = lax.axis_index("s") * NC + lax.axis_index("c")
    lane = lax.broadcasted_iota(jnp.int32, (L,), 0)

    posb = (pos0_v, pos1_v)
    wb = (w0_v, w1_v)
    rows = (rows0_v, rows1_v)
    outs = (out0_v, out1_v)
    gsems = (gsem0, gsem1)
    osems = (osem0, osem1)
    nch = TPW // TCH

    def _gather(ci, b):
        tb = wid * TPW + ci * TCH
        pltpu.sync_copy(pos_hbm.at[pl.ds(2 * tb, 2 * TCH)], posb[b])
        pltpu.sync_copy(wts_hbm.at[pl.ds(2 * tb, 2 * TCH)], wb[b])
        return pltpu.async_copy(ys_hbm.at[posb[b]], rows[b], gsems[b])

    g_next = _gather(0, 0)
    out_pending = [None, None]
    for ci in range(nch):
        b = ci % 2
        g_next.wait()
        if ci + 1 < nch:
            g_next = _gather(ci + 1, 1 - b)
        if out_pending[b] is not None:
            out_pending[b].wait()
            out_pending[b] = None
        rows_v, w_v, out_v = rows[b], wb[b], outs[b]

        def _token(j, _):
            f0 = jnp.full((L,), 2 * j, jnp.int32)
            f1 = jnp.full((L,), 2 * j + 1, jnp.int32)
            w0 = plsc.load_gather(w_v, [f0])
            w1 = plsc.load_gather(w_v, [f1])
            fj = jnp.full((L,), j, jnp.int32)

            def _vec(v, _):
                cols = lane + v * L
                r0 = plsc.load_gather(rows_v, [f0, cols])
                r1 = plsc.load_gather(rows_v, [f1, cols])
                plsc.store_scatter(out_v, [fj, cols], w0 * r0 + w1 * r1)
                return 0
            lax.fori_loop(0, D // L, _vec, 0, unroll=8)
            return 0
        lax.fori_loop(0, TCH, _token, 0)
        tb = wid * TPW + ci * TCH
        out_pending[b] = pltpu.async_copy(
            out_v, out_hbm.at[pl.ds(tb, TCH)], osems[b])
    for d in out_pending:
        if d is not None:
            d.wait()


@functools.lru_cache(maxsize=1)
def _make_sc_kernels():
    mesh = plsc.VectorSubcoreMesh(
        core_axis_name="c", subcore_axis_name="s",
        num_cores=NC, num_subcores=NS)
    sc_params = pltpu.CompilerParams(needs_layout_passes=False)
    dispatch = pl.kernel(
        _sc_dispatch_body,
        out_type=jax.ShapeDtypeStruct((P, D), jnp.float32),
        mesh=mesh,
        compiler_params=sc_params,
        scratch_types=[
            pltpu.VMEM((NK,), jnp.int32),
            pltpu.VMEM((RPW,), jnp.int32),
            pltpu.VMEM((RCH, D), jnp.float32),
            pltpu.VMEM((RCH, D), jnp.float32),
            pltpu.SemaphoreType.DMA,
            pltpu.SemaphoreType.DMA,
            pltpu.SemaphoreType.DMA,
            pltpu.SemaphoreType.DMA,
        ],
    )
    combine = pl.kernel(
        _sc_combine_body,
        out_type=jax.ShapeDtypeStruct((N, D), jnp.float32),
        mesh=mesh,
        compiler_params=sc_params,
        scratch_types=[
            pltpu.VMEM((2 * TCH,), jnp.int32),
            pltpu.VMEM((2 * TCH,), jnp.int32),
            pltpu.VMEM((2 * TCH,), jnp.float32),
            pltpu.VMEM((2 * TCH,), jnp.float32),
            pltpu.VMEM((2 * TCH, D), jnp.float32),
            pltpu.VMEM((2 * TCH, D), jnp.float32),
            pltpu.VMEM((TCH, D), jnp.float32),
            pltpu.VMEM((TCH, D), jnp.float32),
            pltpu.SemaphoreType.DMA,
            pltpu.SemaphoreType.DMA,
            pltpu.SemaphoreType.DMA,
            pltpu.SemaphoreType.DMA,
        ],
    )
    return dispatch, combine


@jax.jit
def kernel(x, Wr, W1, W2):
    xf = x.reshape(N, D)
    _sc_dispatch, _sc_combine = _make_sc_kernels()

    pos, wts, be = pl.pallas_call(
        _router_kernel,
        in_specs=[
            pl.BlockSpec((N, D), lambda: (0, 0)),
            pl.BlockSpec((E, D), lambda: (0, 0)),
        ],
        out_specs=[
            pl.BlockSpec((N, K), lambda: (0, 0)),
            pl.BlockSpec((N, K), lambda: (0, 0)),
            pl.BlockSpec((NBLK, 1), lambda: (0, 0)),
        ],
        out_shape=[
            jax.ShapeDtypeStruct((N, K), jnp.int32),
            jax.ShapeDtypeStruct((N, K), jnp.float32),
            jax.ShapeDtypeStruct((NBLK, 1), jnp.int32),
        ],
    )(xf, Wr)

    pos_flat = pos.reshape(NK)
    xs = _sc_dispatch(xf, pos_flat)

    ys = pl.pallas_call(
        _gmm_kernel,
        grid_spec=pltpu.PrefetchScalarGridSpec(
            num_scalar_prefetch=1,
            grid=(F // BF, NBLK),
            in_specs=[
                pl.BlockSpec((BM, D), lambda f, b, be: (b, 0)),
                pl.BlockSpec((1, BF, D), lambda f, b, be: (be[b, 0], f, 0)),
                pl.BlockSpec((1, D, BF), lambda f, b, be: (be[b, 0], 0, f)),
            ],
            out_specs=pl.BlockSpec((P, D), lambda f, b, be: (0, 0)),
        ),
        out_shape=jax.ShapeDtypeStruct((P, D), jnp.float32),
        compiler_params=pltpu.CompilerParams(
            vmem_limit_bytes=100 * 1024 * 1024),
    )(be, xs, W1, W2)

    out = _sc_combine(ys, pos_flat, wts.reshape(NK))
    return out.reshape(B, T, D)
